# Initial kernel scaffold; baseline (speedup 1.0000x reference)
import jax
import jax.numpy as jnp
from jax.experimental import pallas as pl


def kernel(drug1_id, drug2_id, edge_index, rel_id, ent_emb, edge_w, node_w, comp_w, relparam, edge_bn_g, edge_bn_b, node_bn_g, node_bn_b, comp_bn_g, comp_bn_b):
    raise NotImplementedError("write your pallas kernel here")


# speedup vs baseline: 2.8652x; 2.8652x over previous
import functools
import jax
import jax.numpy as jnp
from jax import lax
from jax.experimental import pallas as pl
from jax.experimental.pallas import tpu as pltpu, tpu_sc as plsc

N = 1559
H = 128
R2 = 172
RPAD = 256
NPAD = 1664
NROW = 2048
E_REAL = 200000
EP = 200704
CH = 128
NSUB = 16
NCORE = 2
EPT = EP // (NSUB * NCORE)
NCH = EPT // CH
EPT_ALL = EP // NSUB
NCH_ALL = EPT_ALL // CH
AH = NPAD // 2
NEG_INF = float("-inf")

_mesh = plsc.VectorSubcoreMesh(core_axis_name="c", subcore_axis_name="s")


def _iota16():
    return lax.broadcasted_iota(jnp.int32, (16,), 0)


def _zero_fill(ref, n):
    def body(j, _):
        ref[pl.ds(j * 16, 16)] = jnp.zeros((16,), jnp.float32)
        return 0
    lax.fori_loop(0, n // 16, body, 0)


def _counts_body(src_hbm, dst_hbm, rel_hbm, a0_hbm, a1_hbm, c_hbm,
                 srcb, dstb, relb, idxv, valv, tmpv, a_sp, c_sp):
    c = lax.axis_index("c")
    s = lax.axis_index("s")

    _zero_fill(tmpv, 8192)
    apt = AH * NPAD // NSUB
    off = 0
    while off < apt:
        sz = min(8192, apt - off)
        pltpu.sync_copy(tmpv.at[pl.ds(0, sz)], a_sp.at[pl.ds(s * apt + off, sz)])
        off += sz
    cpt = NPAD * RPAD // NSUB
    off = 0
    while off < cpt:
        sz = min(8192, cpt - off)
        pltpu.sync_copy(tmpv.at[pl.ds(0, sz)], c_sp.at[pl.ds(s * cpt + off, sz)])
        off += sz
    plsc.subcore_barrier()

    rbase = c * AH

    def chunk(i, _):
        g0 = s * EPT_ALL + i * CH
        pltpu.sync_copy(src_hbm.at[pl.ds(g0, CH)], srcb)
        pltpu.sync_copy(dst_hbm.at[pl.ds(g0, CH)], dstb)
        pltpu.sync_copy(rel_hbm.at[pl.ds(g0, CH)], relb)
        for b in range(CH // 16):
            sl = pl.ds(b * 16, 16)
            d = dstb[sl]
            u = srcb[sl]
            ok = (d >= rbase) & (d < rbase + AH)
            ia = jnp.where(ok, (d - rbase) * NPAD + u, 0)
            idxv[sl] = ia
            valv[sl] = jnp.where(ok, 1.0, 0.0).astype(jnp.float32)
        pltpu.sync_copy(valv, a_sp.at[idxv], add=True)

        @pl.when(c == 0)
        def _():
            for b in range(CH // 16):
                sl = pl.ds(b * 16, 16)
                idxv[sl] = dstb[sl] * RPAD + relb[sl]
                valv[sl] = jnp.ones((16,), jnp.float32)
            pltpu.sync_copy(valv, c_sp.at[idxv], add=True)
        return 0

    lax.fori_loop(0, NCH_ALL, chunk, 0)
    plsc.subcore_barrier()

    def copy_out(sp, hbm, per_tile):
        off2 = 0
        while off2 < per_tile:
            sz = min(8192, per_tile - off2)
            o = s * per_tile + off2
            pltpu.sync_copy(sp.at[pl.ds(o, sz)], tmpv.at[pl.ds(0, sz)])
            pltpu.sync_copy(tmpv.at[pl.ds(0, sz)], hbm.at[pl.ds(o, sz)])
            off2 += sz

    @pl.when(c == 0)
    def _():
        copy_out(a_sp, a0_hbm, apt)
        copy_out(c_sp, c_hbm, cpt)

    @pl.when(c == 1)
    def _():
        copy_out(a_sp, a1_hbm, apt)


_counts_call = pl.kernel(
    _counts_body,
    out_type=(
        jax.ShapeDtypeStruct((AH * NPAD,), jnp.float32),
        jax.ShapeDtypeStruct((AH * NPAD,), jnp.float32),
        jax.ShapeDtypeStruct((NPAD * RPAD,), jnp.float32),
    ),
    mesh=_mesh,
    compiler_params=pltpu.CompilerParams(needs_layout_passes=False),
    scratch_types=[
        pltpu.VMEM((CH,), jnp.int32),
        pltpu.VMEM((CH,), jnp.int32),
        pltpu.VMEM((CH,), jnp.int32),
        pltpu.VMEM((CH,), jnp.int32),
        pltpu.VMEM((CH,), jnp.float32),
        pltpu.VMEM((8192,), jnp.float32),
        pltpu.VMEM_SHARED((AH * NPAD,), jnp.float32),
        pltpu.VMEM_SHARED((NPAD * RPAD,), jnp.float32),
    ],
)


def _pass1_body(src_hbm, dst_hbm, rel_hbm, x_hbm, rp_hbm, sc_hbm, m_hbm,
                srcb, dstb, relb, xs, xd, rl, scb, mv, maccv, slabv, mpart_sp):
    c = lax.axis_index("c")
    s = lax.axis_index("s")

    def initm(j, _):
        mv[pl.ds(j * 16, 16)] = jnp.full((16,), NEG_INF, jnp.float32)
        return 0
    lax.fori_loop(0, NROW // 16, initm, 0)

    def chunk(i, _):
        g0 = (c * NSUB + s) * EPT + i * CH
        pltpu.sync_copy(src_hbm.at[pl.ds(g0, CH)], srcb)
        pltpu.sync_copy(dst_hbm.at[pl.ds(g0, CH)], dstb)
        pltpu.sync_copy(rel_hbm.at[pl.ds(g0, CH)], relb)
        pltpu.sync_copy(x_hbm.at[srcb], xs)
        pltpu.sync_copy(x_hbm.at[dstb], xd)
        pltpu.sync_copy(rp_hbm.at[relb], rl)
        for b in range(CH // 16):
            rows = b * 16 + _iota16()

            def dot_h(h, acc):
                col = jnp.full((16,), h, jnp.int32)
                xsv = plsc.load_gather(xs, [rows, col])
                xdv = plsc.load_gather(xd, [rows, col])
                rlv = plsc.load_gather(rl, [rows, col])
                return acc + xsv * rlv * xdv

            acc = lax.fori_loop(0, H, dot_h, jnp.zeros((16,), jnp.float32))
            scb[pl.ds(b * 16, 16)] = acc

            iota = _iota16()

            def mstep(k, _):
                j16 = b * 16 + ((iota + k) & 15)
                dr = plsc.load_gather(dstb, [j16])
                vr = plsc.load_gather(scb, [j16])
                mask = (dr & 15) == iota
                cur = plsc.load_gather(mv, [dr], mask=mask)
                plsc.store_scatter(mv, [dr], jnp.maximum(cur, vr), mask=mask)
                return 0

            lax.fori_loop(0, 16, mstep, 0)
        pltpu.sync_copy(scb, sc_hbm.at[pl.ds(g0, CH)])
        return 0

    lax.fori_loop(0, NCH, chunk, 0)

    pltpu.sync_copy(mv, mpart_sp.at[s])
    plsc.subcore_barrier()
    pltpu.sync_copy(mpart_sp.at[:, pl.ds(s * 128, 128)], slabv)
    for k in range(8):
        acc = slabv[0, pl.ds(k * 16, 16)]
        for r in range(1, NSUB):
            acc = jnp.maximum(acc, slabv[r, pl.ds(k * 16, 16)])
        maccv[pl.ds(k * 16, 16)] = acc
    pltpu.sync_copy(maccv, m_hbm.at[c, pl.ds(s * 128, 128)])


_pass1_call = pl.kernel(
    _pass1_body,
    out_type=(
        jax.ShapeDtypeStruct((EP,), jnp.float32),
        jax.ShapeDtypeStruct((NCORE, NROW), jnp.float32),
    ),
    mesh=_mesh,
    compiler_params=pltpu.CompilerParams(needs_layout_passes=False),
    scratch_types=[
        pltpu.VMEM((CH,), jnp.int32),
        pltpu.VMEM((CH,), jnp.int32),
        pltpu.VMEM((CH,), jnp.int32),
        pltpu.VMEM((CH, H), jnp.float32),
        pltpu.VMEM((CH, H), jnp.float32),
        pltpu.VMEM((CH, H), jnp.float32),
        pltpu.VMEM((CH,), jnp.float32),
        pltpu.VMEM((NROW,), jnp.float32),
        pltpu.VMEM((128,), jnp.float32),
        pltpu.VMEM((NSUB, 128), jnp.float32),
        pltpu.VMEM_SHARED((NSUB, NROW), jnp.float32),
    ],
)


def _pass2_body(src_hbm, dst_hbm, rel_hbm, x_hbm, rp_hbm, sc_hbm, m_hbm,
                s_hbm, neigh_hbm,
                srcb, dstb, relb, xs, rl, msg, scb, eb, mav, mbv, rowv,
                s_sp, neigh_sp):
    c = lax.axis_index("c")
    s = lax.axis_index("s")

    pltpu.sync_copy(m_hbm.at[0], mav)
    pltpu.sync_copy(m_hbm.at[1], mbv)

    def mmax(j, _):
        sl = pl.ds(j * 16, 16)
        mav[sl] = jnp.maximum(mav[sl], mbv[sl])
        return 0
    lax.fori_loop(0, NROW // 16, mmax, 0)

    def zrow(j, _):
        for k in range(H // 16):
            rowv[j, pl.ds(k * 16, 16)] = jnp.zeros((16,), jnp.float32)
        return 0
    lax.fori_loop(0, 128, zrow, 0)
    _zero_fill(eb, CH)
    pltpu.sync_copy(eb, s_sp.at[pl.ds(s * 128, 128)])
    pltpu.sync_copy(rowv, neigh_sp.at[pl.ds(s * 128, 128)])
    plsc.subcore_barrier()

    def chunk(i, _):
        g0 = (c * NSUB + s) * EPT + i * CH
        pltpu.sync_copy(src_hbm.at[pl.ds(g0, CH)], srcb)
        pltpu.sync_copy(dst_hbm.at[pl.ds(g0, CH)], dstb)
        pltpu.sync_copy(rel_hbm.at[pl.ds(g0, CH)], relb)
        pltpu.sync_copy(sc_hbm.at[pl.ds(g0, CH)], scb)
        pltpu.sync_copy(x_hbm.at[srcb], xs)
        pltpu.sync_copy(rp_hbm.at[relb], rl)
        for b in range(CH // 16):
            sl = pl.ds(b * 16, 16)
            dv = dstb[sl]
            mvv = plsc.load_gather(mav, [dv])
            ev = jnp.exp(scb[sl] - mvv)
            eb[sl] = ev
            rows = b * 16 + _iota16()

            def msg_h(h, _):
                col = jnp.full((16,), h, jnp.int32)
                xsv = plsc.load_gather(xs, [rows, col])
                rlv = plsc.load_gather(rl, [rows, col])
                plsc.store_scatter(msg, [rows, col], xsv * rlv * ev)
                return 0
            lax.fori_loop(0, H, msg_h, 0)
        pltpu.sync_copy(eb, s_sp.at[dstb], add=True)
        pltpu.sync_copy(msg, neigh_sp.at[dstb], add=True)
        return 0

    lax.fori_loop(0, NCH, chunk, 0)
    plsc.subcore_barrier()

    pltpu.sync_copy(s_sp.at[pl.ds(s * 128, 128)], eb)
    pltpu.sync_copy(eb, s_hbm.at[c, pl.ds(s * 128, 128)])
    pltpu.sync_copy(neigh_sp.at[pl.ds(s * 128, 128)], msg)
    pltpu.sync_copy(msg, neigh_hbm.at[c, pl.ds(s * 128, 128)])


_pass2_call = pl.kernel(
    _pass2_body,
    out_type=(
        jax.ShapeDtypeStruct((NCORE, NROW), jnp.float32),
        jax.ShapeDtypeStruct((NCORE, NROW, H), jnp.float32),
    ),
    mesh=_mesh,
    compiler_params=pltpu.CompilerParams(needs_layout_passes=False),
    scratch_types=[
        pltpu.VMEM((CH,), jnp.int32),
        pltpu.VMEM((CH,), jnp.int32),
        pltpu.VMEM((CH,), jnp.int32),
        pltpu.VMEM((CH, H), jnp.float32),
        pltpu.VMEM((CH, H), jnp.float32),
        pltpu.VMEM((CH, H), jnp.float32),
        pltpu.VMEM((CH,), jnp.float32),
        pltpu.VMEM((CH,), jnp.float32),
        pltpu.VMEM((NROW,), jnp.float32),
        pltpu.VMEM((NROW,), jnp.float32),
        pltpu.VMEM((128, H), jnp.float32),
        pltpu.VMEM_SHARED((NROW,), jnp.float32),
        pltpu.VMEM_SHARED((NROW, H), jnp.float32),
    ],
)


def _gather_body(x_hbm, ids_hbm, out_hbm, idxb, rowsv):
    c = lax.axis_index("c")
    s = lax.axis_index("s")
    wid = c * NSUB + s
    for k in range(2):
        base = wid * 256 + k * 128
        pltpu.sync_copy(ids_hbm.at[pl.ds(base, 128)], idxb)
        pltpu.sync_copy(x_hbm.at[idxb], rowsv)
        pltpu.sync_copy(rowsv, out_hbm.at[pl.ds(base, 128)])


_gather_call = pl.kernel(
    _gather_body,
    out_type=jax.ShapeDtypeStruct((8192, H), jnp.float32),
    mesh=_mesh,
    compiler_params=pltpu.CompilerParams(needs_layout_passes=False),
    scratch_types=[
        pltpu.VMEM((128,), jnp.int32),
        pltpu.VMEM((128, H), jnp.float32),
    ],
)


def _dense_body(xb_ref, xf_ref, a_ref, cm_ref, rp_ref, we_ref, wn_ref,
                y1_ref, y2_ref):
    xb = xb_ref[...]
    xf = xf_ref[...]

    def masked_softmax_agg(scores, counts, table):
        neg = jnp.full_like(scores, NEG_INF)
        sm = jnp.where(counts > 0, scores, neg)
        m = jnp.max(sm, axis=1, keepdims=True)
        m = jnp.where(jnp.isfinite(m), m, 0.0)
        w = counts * jnp.exp(sm - m)
        t = jnp.sum(w, axis=1, keepdims=True)
        p = w / (t + 1e-16)
        return lax.dot_general(p, table, (((1,), (0,)), ((), ())),
                               preferred_element_type=jnp.float32, precision=lax.Precision.HIGHEST)

    s2 = lax.dot_general(xb, xf, (((1,), (1,)), ((), ())),
                         preferred_element_type=jnp.float32, precision=lax.Precision.HIGHEST)
    nn = masked_softmax_agg(s2, a_ref[...], xf)
    s1 = lax.dot_general(xb, rp_ref[...], (((1,), (1,)), ((), ())),
                         preferred_element_type=jnp.float32, precision=lax.Precision.HIGHEST)
    ne = masked_softmax_agg(s1, cm_ref[...], rp_ref[...])
    y1_ref[...] = lax.dot_general(ne, we_ref[...], (((1,), (0,)), ((), ())),
                                  preferred_element_type=jnp.float32)
    y2_ref[...] = lax.dot_general(nn, wn_ref[...], (((1,), (0,)), ((), ())),
                                  preferred_element_type=jnp.float32)


def _dense_call(x, a, cm, rp, we, wn):
    grid = NPAD // 128
    return pl.pallas_call(
        _dense_body,
        grid=(grid,),
        in_specs=[
            pl.BlockSpec((128, H), lambda i: (i, 0)),
            pl.BlockSpec((NPAD, H), lambda i: (0, 0)),
            pl.BlockSpec((128, NPAD), lambda i: (i, 0)),
            pl.BlockSpec((128, RPAD), lambda i: (i, 0)),
            pl.BlockSpec((RPAD, H), lambda i: (0, 0)),
            pl.BlockSpec((H, H), lambda i: (0, 0)),
            pl.BlockSpec((H, H), lambda i: (0, 0)),
        ],
        out_specs=[
            pl.BlockSpec((128, H), lambda i: (i, 0)),
            pl.BlockSpec((128, H), lambda i: (i, 0)),
        ],
        out_shape=[
            jax.ShapeDtypeStruct((NPAD, H), jnp.float32),
            jax.ShapeDtypeStruct((NPAD, H), jnp.float32),
        ],
    )(x, x, a, cm, rp, we, wn)


def _combine_body(x_ref, y1_ref, y2_ref, n0_ref, n1_ref, s0_ref, s1_ref,
                  wc_ref, eg_ref, ebb_ref, ng_ref, nb_ref, cg_ref, cb_ref,
                  out_ref):
    rowmask = lax.broadcasted_iota(jnp.int32, (NPAD, H), 0) < N
    denom = (s0_ref[...] + s1_ref[...] + 1e-16)[:, None]
    cn = jnp.where(rowmask, (n0_ref[...] + n1_ref[...]) / denom, 0.0)
    y3 = lax.dot_general(cn, wc_ref[...], (((1,), (0,)), ((), ())),
                         preferred_element_type=jnp.float32)

    def bn_tanh(y, g, b):
        ym = jnp.where(rowmask, y, 0.0)
        mu = jnp.sum(ym, axis=0, keepdims=True) / N
        d = y - mu
        var = jnp.sum(jnp.where(rowmask, d * d, 0.0), axis=0, keepdims=True) / N
        return jnp.tanh(g[None, :] * d / jnp.sqrt(var + 1e-5) + b[None, :])

    o1 = bn_tanh(y1_ref[...], eg_ref[...], ebb_ref[...])
    o2 = bn_tanh(y2_ref[...], ng_ref[...], nb_ref[...])
    o3 = bn_tanh(y3, cg_ref[...], cb_ref[...])
    out_ref[...] = jnp.where(rowmask, x_ref[...] + o1 + o2 + o3, 0.0)


def _combine_call(x, y1, y2, n0, n1, s0, s1, wc, eg, eb, ng, nb, cg, cb):
    return pl.pallas_call(
        _combine_body,
        out_shape=jax.ShapeDtypeStruct((NPAD, H), jnp.float32),
    )(x, y1, y2, n0, n1, s0, s1, wc, eg, eb, ng, nb, cg, cb)


@jax.jit
def kernel(drug1_id, drug2_id, edge_index, rel_id, ent_emb, edge_w, node_w,
           comp_w, relparam, edge_bn_g, edge_bn_b, node_bn_g, node_bn_b,
           comp_bn_g, comp_bn_b):
    pad = EP - E_REAL
    srcp = jnp.concatenate([edge_index[0].astype(jnp.int32),
                            jnp.zeros((pad,), jnp.int32)])
    dstp = jnp.concatenate([edge_index[1].astype(jnp.int32),
                            jnp.full((pad,), N, jnp.int32)])
    relp = jnp.concatenate([rel_id.astype(jnp.int32),
                            jnp.zeros((pad,), jnp.int32)])
    x = jnp.pad(ent_emb, ((0, NPAD - N), (0, 0)))
    rp_pad = jnp.pad(relparam, ((0, RPAD - R2), (0, 0)))

    a0, a1, cm = _counts_call(srcp, dstp, relp)
    a = jnp.concatenate([a0.reshape(AH, NPAD), a1.reshape(AH, NPAD)], axis=0)
    cm = cm.reshape(NPAD, RPAD)

    for l in range(edge_w.shape[0]):
        sc, m = _pass1_call(srcp, dstp, relp, x, relparam)
        sv, neigh = _pass2_call(srcp, dstp, relp, x, relparam, sc, m)
        y1, y2 = _dense_call(x, a, cm, rp_pad, edge_w[l], node_w[l])
        x = _combine_call(x, y1, y2, neigh[0, :NPAD], neigh[1, :NPAD],
                          sv[0, :NPAD], sv[1, :NPAD], comp_w[l],
                          edge_bn_g[l], edge_bn_b[l], node_bn_g[l],
                          node_bn_b[l], comp_bn_g[l], comp_bn_b[l])

    ids = jnp.concatenate([drug1_id, drug2_id]).astype(jnp.int32)
    rows = _gather_call(x, ids)
    return rows[:4096], rows[4096:]
